# baseline (speedup 1.0000x reference)
import functools
import math

import jax
import jax.numpy as jnp
from jax import lax
from jax.experimental import pallas as pl
from jax.experimental.pallas import tpu as pltpu
from jax.experimental.pallas import tpu_sc as plsc

VOCAB = 100000
HIDDEN = 1024
EPS = 1e-06
LANES = 16
SCALE = math.sqrt(HIDDEN)
NBUF = 4
PREFETCH = 2
CHUNK = 16


def _make_kernel(num_tokens):
    info = plsc.get_sparse_core_info()
    nw = info.num_cores * info.num_subcores
    assert num_tokens % nw == 0
    tok_per_w = num_tokens // nw
    assert tok_per_w % CHUNK == 0
    n_chunks = tok_per_w // CHUNK
    jvec = HIDDEN // LANES

    mesh = plsc.VectorSubcoreMesh(core_axis_name="c", subcore_axis_name="s")

    @functools.partial(
        pl.kernel,
        mesh=mesh,
        out_type=jax.ShapeDtypeStruct((num_tokens, HIDDEN), jnp.float32),
        scratch_types=[
            pltpu.VMEM((n_chunks, CHUNK), jnp.int32),
            pltpu.VMEM((NBUF, CHUNK, HIDDEN), jnp.float32),
            pltpu.VMEM((HIDDEN,), jnp.float32),
            pltpu.SemaphoreType.DMA((NBUF,)),
            pltpu.SemaphoreType.DMA((NBUF,)),
        ],
    )
    def k(ids_hbm, table_hbm, nwt_hbm, out_hbm, idx_v, buf_v, nwt_v, gsem, ssem):
        wid = lax.axis_index("s") * info.num_cores + lax.axis_index("c")
        base = wid * tok_per_w
        pltpu.sync_copy(ids_hbm.at[pl.ds(wid * n_chunks, n_chunks)], idx_v)
        pltpu.sync_copy(nwt_hbm, nwt_v)

        def gstart(c, b):
            pltpu.async_copy(
                table_hbm.at[idx_v.at[c]], buf_v.at[b], gsem.at[b]
            )

        def gwait(c, b):
            pltpu.make_async_copy(
                table_hbm.at[idx_v.at[c]], buf_v.at[b], gsem.at[b]
            ).wait()

        def sstart(c, b):
            pltpu.async_copy(
                buf_v.at[b],
                out_hbm.at[pl.ds(base + c * CHUNK, CHUNK)],
                ssem.at[b],
            )

        def swait(c, b):
            pltpu.make_async_copy(
                buf_v.at[b],
                out_hbm.at[pl.ds(base + c * CHUNK, CHUNK)],
                ssem.at[b],
            ).wait()

        lane_iota = lax.iota(jnp.int32, LANES)

        def compute(b):
            def _row_ss(r2, ss):
                for dr in range(2):
                    r = r2 * 2 + dr
                    accs = [jnp.zeros((LANES,), jnp.float32) for _ in range(4)]
                    for j in range(jvec):
                        v = buf_v[b, r, pl.ds(j * LANES, LANES)]
                        accs[j % 4] = accs[j % 4] + v * v
                    acc = (accs[0] + accs[1]) + (accs[2] + accs[3])
                    for s in (8, 4, 2, 1):
                        perm = lane_iota ^ s
                        acc = acc + acc.at[perm].get(mode="promise_in_bounds")
                    ss = jnp.where(lane_iota == r, acc, ss)
                return ss

            ss = lax.fori_loop(
                0, LANES // 2, _row_ss, jnp.zeros((LANES,), jnp.float32)
            )

            vv = ss + EPS
            i = lax.bitcast_convert_type(vv, jnp.int32)
            i = jnp.int32(0x5F3759DF) - (i >> 1)
            y = lax.bitcast_convert_type(i, jnp.float32)
            for _ in range(3):
                y = y * (1.5 - 0.5 * vv * y * y)
            y = y * SCALE

            ys = [
                y.at[jnp.full((LANES,), r, jnp.int32)].get(
                    mode="promise_in_bounds"
                )
                for r in range(LANES)
            ]

            def _col(j4, _):
                for dj in range(4):
                    j = j4 * 4 + dj
                    js = pl.ds(j * LANES, LANES)
                    w = nwt_v[js]
                    for r in range(LANES):
                        buf_v[b, r, js] = buf_v[b, r, js] * ys[r] * w
                return 0

            lax.fori_loop(0, jvec // 4, _col, 0)

        for p in range(PREFETCH):
            gstart(p, p)

        def chunk_body(c, _):
            b = lax.rem(c, NBUF)
            nb = lax.rem(c + PREFETCH, NBUF)

            @pl.when(
                jnp.logical_and(c >= NBUF - PREFETCH, c + PREFETCH < n_chunks)
            )
            def _():
                swait(c + PREFETCH - NBUF, nb)

            @pl.when(c + PREFETCH < n_chunks)
            def _():
                gstart(c + PREFETCH, nb)

            gwait(c, b)
            compute(b)
            sstart(c, b)
            return 0

        lax.fori_loop(0, n_chunks, chunk_body, 0)
        for t in range(NBUF):
            c = n_chunks - NBUF + t
            swait(c, lax.rem(jnp.int32(c), NBUF))

    return k


def kernel(input_ids, embed_weight, norm_weight):
    b, t = input_ids.shape
    ids = input_ids.reshape(b * t // CHUNK, CHUNK)
    k = _make_kernel(b * t)
    out = k(ids, embed_weight, norm_weight)
    return out.reshape(b, t, HIDDEN)


# speedup vs baseline: 1.0949x; 1.0949x over previous
import functools
import math

import jax
import jax.numpy as jnp
from jax import lax
from jax.experimental import pallas as pl
from jax.experimental.pallas import tpu as pltpu
from jax.experimental.pallas import tpu_sc as plsc

VOCAB = 100000
HIDDEN = 1024
EPS = 1e-06
LANES = 16
SCALE = math.sqrt(HIDDEN)
NBUF = 4
PREFETCH = 2
CHUNK = 16


def _make_kernel(num_tokens):
    info = plsc.get_sparse_core_info()
    nw = info.num_cores * info.num_subcores
    assert num_tokens % nw == 0
    tok_per_w = num_tokens // nw
    assert tok_per_w % CHUNK == 0
    n_chunks = tok_per_w // CHUNK
    jvec = HIDDEN // LANES

    mesh = plsc.VectorSubcoreMesh(core_axis_name="c", subcore_axis_name="s")

    @functools.partial(
        pl.kernel,
        mesh=mesh,
        out_type=jax.ShapeDtypeStruct((num_tokens, HIDDEN), jnp.float32),
        scratch_types=[
            pltpu.VMEM((n_chunks, CHUNK), jnp.int32),
            pltpu.VMEM((NBUF, CHUNK, HIDDEN), jnp.float32),
            pltpu.VMEM((HIDDEN,), jnp.float32),
            pltpu.SemaphoreType.DMA((NBUF,)),
            pltpu.SemaphoreType.DMA((NBUF,)),
        ],
    )
    def k(ids_hbm, table_hbm, nwt_hbm, out_hbm, idx_v, buf_v, nwt_v, gsem, ssem):
        wid = lax.axis_index("s") * info.num_cores + lax.axis_index("c")
        base = wid * tok_per_w
        pltpu.sync_copy(ids_hbm.at[pl.ds(wid * n_chunks, n_chunks)], idx_v)
        pltpu.sync_copy(nwt_hbm, nwt_v)

        def gstart(c, b):
            pltpu.async_copy(
                table_hbm.at[idx_v.at[c]], buf_v.at[b], gsem.at[b]
            )

        def gwait(c, b):
            pltpu.make_async_copy(
                table_hbm.at[idx_v.at[c]], buf_v.at[b], gsem.at[b]
            ).wait()

        def sstart(c, b):
            pltpu.async_copy(
                buf_v.at[b],
                out_hbm.at[pl.ds(base + c * CHUNK, CHUNK)],
                ssem.at[b],
            )

        def swait(c, b):
            pltpu.make_async_copy(
                buf_v.at[b],
                out_hbm.at[pl.ds(base + c * CHUNK, CHUNK)],
                ssem.at[b],
            ).wait()

        lane_iota = lax.iota(jnp.int32, LANES)

        def compute(b):
            def _row_ss(r, ss):
                accs = [jnp.zeros((LANES,), jnp.float32) for _ in range(4)]
                for j in range(jvec):
                    v = buf_v[b, r, pl.ds(j * LANES, LANES)]
                    accs[j % 4] = accs[j % 4] + v * v
                acc = (accs[0] + accs[1]) + (accs[2] + accs[3])
                for s in (8, 4, 2, 1):
                    perm = lane_iota ^ s
                    acc = acc + acc.at[perm].get(mode="promise_in_bounds")
                return jnp.where(lane_iota == r, acc, ss)

            ss = lax.fori_loop(
                0, LANES, _row_ss, jnp.zeros((LANES,), jnp.float32)
            )

            vv = ss + EPS
            i = lax.bitcast_convert_type(vv, jnp.int32)
            i = jnp.int32(0x5F3759DF) - (i >> 1)
            y = lax.bitcast_convert_type(i, jnp.float32)
            for _ in range(3):
                y = y * (1.5 - 0.5 * vv * y * y)
            y = y * SCALE

            ys = [
                y.at[jnp.full((LANES,), r, jnp.int32)].get(
                    mode="promise_in_bounds"
                )
                for r in range(LANES)
            ]

            def _col(j2, _):
                for dj in range(2):
                    j = j2 * 2 + dj
                    js = pl.ds(j * LANES, LANES)
                    w = nwt_v[js]
                    for r in range(LANES):
                        buf_v[b, r, js] = buf_v[b, r, js] * ys[r] * w
                return 0

            lax.fori_loop(0, jvec // 2, _col, 0)

        for p in range(PREFETCH):
            gstart(p, p)

        def chunk_body(c, _):
            b = lax.rem(c, NBUF)
            nb = lax.rem(c + PREFETCH, NBUF)

            @pl.when(
                jnp.logical_and(c >= NBUF - PREFETCH, c + PREFETCH < n_chunks)
            )
            def _():
                swait(c + PREFETCH - NBUF, nb)

            @pl.when(c + PREFETCH < n_chunks)
            def _():
                gstart(c + PREFETCH, nb)

            gwait(c, b)
            compute(b)
            sstart(c, b)
            return 0

        lax.fori_loop(0, n_chunks, chunk_body, 0)
        for t in range(NBUF):
            c = n_chunks - NBUF + t
            swait(c, lax.rem(jnp.int32(c), NBUF))

    return k


def kernel(input_ids, embed_weight, norm_weight):
    b, t = input_ids.shape
    ids = input_ids.reshape(b * t // CHUNK, CHUNK)
    k = _make_kernel(b * t)
    out = k(ids, embed_weight, norm_weight)
    return out.reshape(b, t, HIDDEN)
